# baseline (speedup 1.0000x reference)
import functools

import jax
import jax.numpy as jnp
from jax import lax
from jax.experimental import pallas as pl
from jax.experimental.pallas import tpu as pltpu
from jax.experimental.pallas import tpu_sc as plsc

VOCAB = 23236
EMBED_DIM = 1728
BATCH = 64
SEQ = 512
N = BATCH * SEQ

NC = 2
NS = 16
NW = NC * NS
PER_W = N // NW
ROWS_W = PER_W // SEQ
CHUNK = 16
NCHUNK = PER_W // CHUNK
CPR = SEQ // CHUNK
NB = 4
NPIECE = EMBED_DIM // 128
EDGE = EMBED_DIM - NPIECE * 128
NP1 = NPIECE + 1


def _gather_body(idx_hbm, table_hbm, tail_hbm, out_hbm, eout_hbm, idx_v, *rest):
    bufs = rest[: NB * NP1]
    gsems = rest[NB * NP1 : NB * NP1 + NB]
    wsems = rest[NB * NP1 + NB :]
    wid = lax.axis_index("s") * NC + lax.axis_index("c")
    row0 = wid * ROWS_W
    band = row0 // 8
    sub = row0 % 8
    pltpu.sync_copy(idx_hbm.at[pl.ds(band * 8, 8)], idx_v)

    def idx_slc(c):
        return idx_v.at[sub + c // CPR, pl.ds((c % CPR) * CHUNK, CHUNK)]

    def gathers(c, b):
        idx = idx_slc(c)
        cps = []
        for p in range(NPIECE):
            cps.append(
                pltpu.make_async_copy(
                    table_hbm.at[idx, pl.ds(p * 128, 128)],
                    bufs[b * NP1 + p],
                    gsems[b],
                )
            )
        cps.append(
            pltpu.make_async_copy(tail_hbm.at[idx], bufs[b * NP1 + NPIECE], gsems[b])
        )
        return cps

    def writes(c, b):
        r = row0 + c // CPR
        col = (c % CPR) * CHUNK
        cps = []
        for p in range(NPIECE):
            cps.append(
                pltpu.make_async_copy(
                    bufs[b * NP1 + p],
                    out_hbm.at[r, pl.ds(col, CHUNK), pl.ds(p * 128, 128)],
                    wsems[b],
                )
            )
        cps.append(
            pltpu.make_async_copy(
                bufs[b * NP1 + NPIECE],
                eout_hbm.at[r, pl.ds(col, CHUNK)],
                wsems[b],
            )
        )
        return cps

    for cp in gathers(0, 0):
        cp.start()

    @pl.loop(0, NCHUNK, step=NB)
    def _(c0):
        for b in range(NB):
            c = c0 + b
            for cp in gathers(c, b):
                cp.wait()
            nb = (b + 1) % NB

            @pl.when(c + 1 < NCHUNK)
            def _():
                @pl.when(c + 1 >= NB)
                def _():
                    for cp in writes(c + 1 - NB, nb):
                        cp.wait()

                for cp in gathers(c + 1, nb):
                    cp.start()

            for cp in writes(c, b):
                cp.start()

    for c in range(NCHUNK - NB, NCHUNK):
        for cp in writes(c, c % NB):
            cp.wait()


_gather = pl.kernel(
    _gather_body,
    out_type=(
        jax.ShapeDtypeStruct((BATCH, SEQ, EMBED_DIM), jnp.float32),
        jax.ShapeDtypeStruct((BATCH, SEQ, 128), jnp.float32),
    ),
    mesh=plsc.VectorSubcoreMesh(core_axis_name="c", subcore_axis_name="s"),
    scratch_types=[
        pltpu.VMEM((8, SEQ), jnp.int32),
        *[pltpu.VMEM((CHUNK, 128), jnp.float32) for _ in range(NB * NP1)],
        *[pltpu.SemaphoreType.DMA for _ in range(2 * NB)],
    ],
    compiler_params=pltpu.CompilerParams(use_tc_tiling_on_sc=True),
)


@jax.jit
def kernel(inputs, table):
    tail = jnp.pad(table[:, NPIECE * 128 :], ((0, 0), (0, 128 - EDGE)))
    main, edge = _gather(inputs.astype(jnp.int32), table, tail)
    return lax.dynamic_update_slice(
        main, edge[:, :, :EDGE], (0, 0, NPIECE * 128)
    )


# speedup vs baseline: 1.0120x; 1.0120x over previous
import functools

import jax
import jax.numpy as jnp
from jax import lax
from jax.experimental import pallas as pl
from jax.experimental.pallas import tpu as pltpu
from jax.experimental.pallas import tpu_sc as plsc

VOCAB = 23236
EMBED_DIM = 1728
BATCH = 64
SEQ = 512
N = BATCH * SEQ

NC = 2
NS = 16
NW = NC * NS
PER_W = N // NW
ROWS_W = PER_W // SEQ
CHUNK = 32
NCHUNK = PER_W // CHUNK
CPR = SEQ // CHUNK
NB = 2
NPIECE = EMBED_DIM // 128
EDGE = EMBED_DIM - NPIECE * 128
PIECES = [(0, 512), (512, 512), (1024, 512), (1536, 128)]
NP1 = len(PIECES) + 1


def _gather_body(idx_hbm, table_hbm, tail_hbm, out_hbm, eout_hbm, idx_v, *rest):
    bufs = rest[: NB * NP1]
    gsems = rest[NB * NP1 : NB * NP1 + NB]
    wsems = rest[NB * NP1 + NB :]
    wid = lax.axis_index("s") * NC + lax.axis_index("c")
    row0 = wid * ROWS_W
    band = row0 // 8
    sub = row0 % 8
    pltpu.sync_copy(idx_hbm.at[pl.ds(band * 8, 8)], idx_v)

    def idx_slc(c):
        return idx_v.at[sub + c // CPR, pl.ds((c % CPR) * CHUNK, CHUNK)]

    def gathers(c, b):
        idx = idx_slc(c)
        cps = []
        for p, (off, width) in enumerate(PIECES):
            cps.append(
                pltpu.make_async_copy(
                    table_hbm.at[idx, pl.ds(off, width)],
                    bufs[b * NP1 + p],
                    gsems[b],
                )
            )
        cps.append(
            pltpu.make_async_copy(
                tail_hbm.at[idx], bufs[b * NP1 + len(PIECES)], gsems[b]
            )
        )
        return cps

    def writes(c, b):
        r = row0 + c // CPR
        col = (c % CPR) * CHUNK
        cps = []
        for p, (off, width) in enumerate(PIECES):
            cps.append(
                pltpu.make_async_copy(
                    bufs[b * NP1 + p],
                    out_hbm.at[r, pl.ds(col, CHUNK), pl.ds(off, width)],
                    wsems[b],
                )
            )
        cps.append(
            pltpu.make_async_copy(
                bufs[b * NP1 + len(PIECES)],
                eout_hbm.at[r, pl.ds(col, CHUNK)],
                wsems[b],
            )
        )
        return cps

    for cp in gathers(0, 0):
        cp.start()

    @pl.loop(0, NCHUNK, step=NB)
    def _(c0):
        for b in range(NB):
            c = c0 + b
            for cp in gathers(c, b):
                cp.wait()
            nb = (b + 1) % NB

            @pl.when(c + 1 < NCHUNK)
            def _():
                @pl.when(c + 1 >= NB)
                def _():
                    for cp in writes(c + 1 - NB, nb):
                        cp.wait()

                for cp in gathers(c + 1, nb):
                    cp.start()

            for cp in writes(c, b):
                cp.start()

    for c in range(NCHUNK - NB, NCHUNK):
        for cp in writes(c, c % NB):
            cp.wait()


_gather = pl.kernel(
    _gather_body,
    out_type=(
        jax.ShapeDtypeStruct((BATCH, SEQ, EMBED_DIM), jnp.float32),
        jax.ShapeDtypeStruct((BATCH, SEQ, 128), jnp.float32),
    ),
    mesh=plsc.VectorSubcoreMesh(core_axis_name="c", subcore_axis_name="s"),
    scratch_types=[
        pltpu.VMEM((8, SEQ), jnp.int32),
        *[
            pltpu.VMEM((CHUNK, w), jnp.float32)
            for _ in range(NB)
            for w in [*(w for _, w in PIECES), 128]
        ],
        *[pltpu.SemaphoreType.DMA for _ in range(2 * NB)],
    ],
    compiler_params=pltpu.CompilerParams(use_tc_tiling_on_sc=True),
)


@jax.jit
def kernel(inputs, table):
    tail = jnp.pad(table[:, NPIECE * 128 :], ((0, 0), (0, 128 - EDGE)))
    main, edge = _gather(inputs.astype(jnp.int32), table, tail)
    return lax.dynamic_update_slice(
        main, edge[:, :, :EDGE], (0, 0, NPIECE * 128)
    )


# speedup vs baseline: 1.0157x; 1.0036x over previous
import jax
import jax.numpy as jnp
from jax import lax
from jax.experimental import pallas as pl
from jax.experimental.pallas import tpu as pltpu
from jax.experimental.pallas import tpu_sc as plsc

VOCAB = 23236
EMBED_DIM = 1728
BATCH = 64
SEQ = 512
N = BATCH * SEQ

NC = 2
NS = 16
NW = NC * NS
PER_W = N // NW
ROWS_W = PER_W // SEQ
CHUNK = 32
NCHUNK = PER_W // CHUNK
CPR = SEQ // CHUNK
NB = 2
NPIECE = EMBED_DIM // 128
EDGE = EMBED_DIM - NPIECE * 128
PIECES = [(0, 512), (512, 512), (1024, 512), (1536, 128)]
NP1 = len(PIECES) + 1


def _gather_body(idx_hbm, table_hbm, tail_hbm, out_hbm, eout_hbm, idx_v, *rest):
    bufs = rest[: NB * NP1]
    gsems = rest[NB * NP1 : NB * NP1 + NB]
    wsems = rest[NB * NP1 + NB :]
    wid = lax.axis_index("s") * NC + lax.axis_index("c")
    row0 = wid * ROWS_W
    band = row0 // 8
    sub = row0 % 8
    pltpu.sync_copy(idx_hbm.at[pl.ds(band * 8, 8)], idx_v)

    def idx_slc(c):
        return idx_v.at[sub + c // CPR, pl.ds((c % CPR) * CHUNK, CHUNK)]

    def gathers(c, b):
        idx = idx_slc(c)
        cps = []
        for p, (off, width) in enumerate(PIECES):
            cps.append(
                pltpu.make_async_copy(
                    table_hbm.at[idx, pl.ds(off, width)],
                    bufs[b * NP1 + p],
                    gsems[b],
                )
            )
        cps.append(
            pltpu.make_async_copy(
                tail_hbm.at[idx], bufs[b * NP1 + len(PIECES)], gsems[b]
            )
        )
        return cps

    def writes(c, b):
        r = row0 + c // CPR
        col = (c % CPR) * CHUNK
        cps = []
        for p, (off, width) in enumerate(PIECES):
            cps.append(
                pltpu.make_async_copy(
                    bufs[b * NP1 + p],
                    out_hbm.at[r, pl.ds(col, CHUNK), pl.ds(off, width)],
                    wsems[b],
                )
            )
        cps.append(
            pltpu.make_async_copy(
                bufs[b * NP1 + len(PIECES)],
                eout_hbm.at[r, pl.ds(col, CHUNK)],
                wsems[b],
            )
        )
        return cps

    for cp in gathers(0, 0):
        cp.start()

    @pl.loop(0, NCHUNK, step=NB)
    def _(c0):
        for b in range(NB):
            c = c0 + b
            for cp in gathers(c, b):
                cp.wait()
            nb = (b + 1) % NB

            @pl.when(c + 1 < NCHUNK)
            def _():
                @pl.when(c + 1 >= NB)
                def _():
                    for cp in writes(c + 1 - NB, nb):
                        cp.wait()

                for cp in gathers(c + 1, nb):
                    cp.start()

            for cp in writes(c, b):
                cp.start()

    for c in range(NCHUNK - NB, NCHUNK):
        for cp in writes(c, c % NB):
            cp.wait()


_gather = pl.kernel(
    _gather_body,
    out_type=(
        jax.ShapeDtypeStruct((BATCH, SEQ, EMBED_DIM), jnp.float32),
        jax.ShapeDtypeStruct((BATCH, SEQ, 128), jnp.float32),
    ),
    mesh=plsc.VectorSubcoreMesh(core_axis_name="c", subcore_axis_name="s"),
    scratch_types=[
        pltpu.VMEM((8, SEQ), jnp.int32),
        *[
            pltpu.VMEM((CHUNK, w), jnp.float32)
            for _ in range(NB)
            for w in [*(w for _, w in PIECES), 128]
        ],
        *[pltpu.SemaphoreType.DMA for _ in range(2 * NB)],
    ],
    compiler_params=pltpu.CompilerParams(use_tc_tiling_on_sc=True),
)


@jax.jit
def kernel(inputs, table):
    tail = jnp.pad(table[:, NPIECE * 128 :], ((0, 0), (0, 128 - EDGE)))
    main, edge = _gather(inputs.astype(jnp.int32), table, tail)
    return lax.dynamic_update_slice(
        main, edge[:, :, :EDGE], (0, 0, NPIECE * 128)
    )
